# baseline (speedup 1.0000x reference)
import functools

import jax
import jax.numpy as jnp
from jax import lax
from jax.experimental import pallas as pl
from jax.experimental.pallas import tpu as pltpu
from jax.experimental.pallas import tpu_sc as plsc

_VOCAB, _DIM, _B, _L = 32320, 1024, 1024, 50
_LP = 56
_NC, _NS = 2, 16
_NW = _NC * _NS
_PER_W = _B // _NW

_mesh = plsc.VectorSubcoreMesh(core_axis_name="c", subcore_axis_name="s")


@functools.partial(
    pl.kernel,
    mesh=_mesh,
    out_type=jax.ShapeDtypeStruct((_B, _LP, _DIM), jnp.float32),
    scratch_types=[
        pltpu.VMEM((_PER_W, _LP), jnp.int32),
        pltpu.VMEM((_LP, _DIM), jnp.float32),
        pltpu.VMEM((_LP, _DIM), jnp.float32),
        pltpu.SemaphoreType.DMA,
        pltpu.SemaphoreType.DMA,
    ],
)
def _embed_gather(idx_hbm, table_hbm, out_hbm, idx_v, buf0, buf1, sem0, sem1):
    wid = lax.axis_index("s") * _NC + lax.axis_index("c")
    base = wid * _PER_W
    pltpu.sync_copy(idx_hbm.at[wid], idx_v)

    pltpu.async_copy(table_hbm.at[idx_v.at[0]], buf0, sem0)
    pltpu.async_copy(table_hbm.at[idx_v.at[1]], buf1, sem1)

    def body(i, carry):
        r0 = 2 * i
        pltpu.make_async_copy(table_hbm.at[idx_v.at[r0]], buf0, sem0).wait()
        pltpu.sync_copy(buf0, out_hbm.at[base + r0])
        pltpu.async_copy(table_hbm.at[idx_v.at[r0 + 2]], buf0, sem0)
        r1 = r0 + 1
        pltpu.make_async_copy(table_hbm.at[idx_v.at[r1]], buf1, sem1).wait()
        pltpu.sync_copy(buf1, out_hbm.at[base + r1])
        pltpu.async_copy(table_hbm.at[idx_v.at[r1 + 2]], buf1, sem1)
        return carry

    lax.fori_loop(0, _PER_W // 2 - 1, body, 0)

    r0 = _PER_W - 2
    pltpu.make_async_copy(table_hbm.at[idx_v.at[r0]], buf0, sem0).wait()
    pltpu.sync_copy(buf0, out_hbm.at[base + r0])
    r1 = _PER_W - 1
    pltpu.make_async_copy(table_hbm.at[idx_v.at[r1]], buf1, sem1).wait()
    pltpu.sync_copy(buf1, out_hbm.at[base + r1])


def kernel(src, src_length, tgt_input, embed_weight):
    idx = jnp.pad(src, ((0, 0), (0, _LP - _L))).reshape(_NW, _PER_W, _LP)
    out = _embed_gather(idx, embed_weight)
    return out[:, :_L, :], src_length, tgt_input


# speedup vs baseline: 2.3004x; 2.3004x over previous
import functools

import jax
import jax.numpy as jnp
from jax import lax
from jax.experimental import pallas as pl
from jax.experimental.pallas import tpu as pltpu
from jax.experimental.pallas import tpu_sc as plsc

_VOCAB, _DIM, _B, _L = 32320, 1024, 1024, 50
_LP = 56
_NC, _NS = 2, 16
_NW = _NC * _NS
_PER_W = _B // _NW

_mesh = plsc.VectorSubcoreMesh(core_axis_name="c", subcore_axis_name="s")


@functools.partial(
    pl.kernel,
    mesh=_mesh,
    out_type=jax.ShapeDtypeStruct((_B, _LP, _DIM), jnp.float32),
    scratch_types=[
        pltpu.VMEM((_PER_W, _LP), jnp.int32),
        pltpu.VMEM((_LP, _DIM), jnp.float32),
        pltpu.VMEM((_LP, _DIM), jnp.float32),
        pltpu.SemaphoreType.DMA,
        pltpu.SemaphoreType.DMA,
    ],
)
def _embed_gather(idx_hbm, table_hbm, out_hbm, idx_v, buf0, buf1, sem0, sem1):
    wid = lax.axis_index("s") * _NC + lax.axis_index("c")
    base = wid * _PER_W
    pltpu.sync_copy(idx_hbm.at[wid], idx_v)

    pltpu.async_copy(table_hbm.at[idx_v.at[0]], buf0, sem0)
    pltpu.async_copy(table_hbm.at[idx_v.at[1]], buf1, sem1)

    def body(i, carry):
        r0 = 2 * i
        pltpu.make_async_copy(table_hbm.at[idx_v.at[r0]], buf0, sem0).wait()
        pltpu.sync_copy(buf0, out_hbm.at[base + r0])
        pltpu.async_copy(table_hbm.at[idx_v.at[r0 + 2]], buf0, sem0)
        r1 = r0 + 1
        pltpu.make_async_copy(table_hbm.at[idx_v.at[r1]], buf1, sem1).wait()
        pltpu.sync_copy(buf1, out_hbm.at[base + r1])
        pltpu.async_copy(table_hbm.at[idx_v.at[r1 + 2]], buf1, sem1)
        return carry

    lax.fori_loop(0, _PER_W // 2 - 1, body, 0)

    r0 = _PER_W - 2
    pltpu.make_async_copy(table_hbm.at[idx_v.at[r0]], buf0, sem0).wait()
    pltpu.sync_copy(buf0, out_hbm.at[base + r0])
    r1 = _PER_W - 1
    pltpu.make_async_copy(table_hbm.at[idx_v.at[r1]], buf1, sem1).wait()
    pltpu.sync_copy(buf1, out_hbm.at[base + r1])


def kernel(src, src_length, tgt_input, embed_weight):
    idx = jnp.concatenate([src, src[:, : _LP - _L]], axis=1)
    idx = idx.reshape(_NW, _PER_W, _LP)
    out = _embed_gather(idx, embed_weight)
    return out[:, :_L, :], src_length, tgt_input
